# baseline (speedup 1.0000x reference)
import jax
import jax.numpy as jnp
from jax import lax
from jax.experimental import pallas as pl
from jax.experimental.pallas import tpu as pltpu
from jax.experimental.pallas import tpu_sc as plsc

EMB_DIM = 128
HID_DIM = 256
BATCH = 16384

NUM_CORES = 2
NUM_SUBCORES = 16
NUM_WORKERS = NUM_CORES * NUM_SUBCORES

NCHUNK = 2
CHUNK = BATCH // NCHUNK
B_PER_W = CHUNK // NUM_WORKERS

BB = 4096
BLOCKS_PER_CHUNK = CHUNK // BB


def _gather_body(idx_hbm, table_hbm, out_hbm, idx_v, rows_v, sem):
    wid = lax.axis_index("s") * NUM_CORES + lax.axis_index("c")
    base = wid * B_PER_W
    pltpu.sync_copy(idx_hbm.at[pl.ds(base, B_PER_W)], idx_v)
    pltpu.async_copy(table_hbm.at[idx_v], rows_v, sem).wait()
    pltpu.sync_copy(rows_v, out_hbm.at[pl.ds(base, B_PER_W)])


def _fc_ln(x, w, b, g, be):
    h = jnp.dot(x, w, preferred_element_type=jnp.float32)
    h = jnp.maximum(h + b, 0.0)
    mean = jnp.mean(h, axis=-1, keepdims=True)
    var = jnp.mean(jnp.square(h - mean), axis=-1, keepdims=True)
    h_hat = (h - mean) * lax.rsqrt(var + 1e-5)
    return h_hat * g + be


def _fc_first_body(x_ref, w_ref, b_ref, g_ref, be_ref, o_ref):
    o_ref[...] = _fc_ln(x_ref[...], w_ref[...], b_ref[...], g_ref[...],
                        be_ref[...])


def _fc_second_body(x_ref, w_ref, b_ref, g_ref, be_ref, buf_ref, o_ref):
    del buf_ref
    o_ref[...] = _fc_ln(x_ref[...], w_ref[...], b_ref[...], g_ref[...],
                        be_ref[...])


def _make_gather():
    return pl.kernel(
        _gather_body,
        mesh=plsc.VectorSubcoreMesh(core_axis_name="c", subcore_axis_name="s"),
        out_type=jax.ShapeDtypeStruct((CHUNK, EMB_DIM), jnp.float32),
        scratch_types=[
            pltpu.VMEM((B_PER_W,), jnp.int32),
            pltpu.VMEM((B_PER_W, EMB_DIM), jnp.float32),
            pltpu.SemaphoreType.DMA,
        ],
    )


def kernel(item_input, table, W, b, gamma, beta):
    idx = item_input.astype(jnp.int32)
    b2 = b.reshape(1, HID_DIM)
    g2 = gamma.reshape(1, HID_DIM)
    be2 = beta.reshape(1, HID_DIM)

    gather = _make_gather()
    x0 = gather(idx[:CHUNK], table)
    x1 = gather(idx[CHUNK:], table)

    w_spec = pl.BlockSpec((EMB_DIM, HID_DIM), lambda i: (0, 0))
    v_spec = pl.BlockSpec((1, HID_DIM), lambda i: (0, 0))
    x_spec = pl.BlockSpec((BB, EMB_DIM), lambda i: (i, 0))

    out_a = pl.pallas_call(
        _fc_first_body,
        grid=(BLOCKS_PER_CHUNK,),
        in_specs=[x_spec, w_spec, v_spec, v_spec, v_spec],
        out_specs=pl.BlockSpec((BB, HID_DIM), lambda i: (i, 0)),
        out_shape=jax.ShapeDtypeStruct((BATCH, HID_DIM), jnp.float32),
    )(x0, W, b2, g2, be2)

    out = pl.pallas_call(
        _fc_second_body,
        grid=(BLOCKS_PER_CHUNK,),
        in_specs=[x_spec, w_spec, v_spec, v_spec, v_spec,
                  pl.BlockSpec(memory_space=pl.ANY)],
        out_specs=pl.BlockSpec(
            (BB, HID_DIM), lambda i: (i + BLOCKS_PER_CHUNK, 0)),
        out_shape=jax.ShapeDtypeStruct((BATCH, HID_DIM), jnp.float32),
        input_output_aliases={5: 0},
    )(x1, W, b2, g2, be2, out_a)
    return out


# speedup vs baseline: 1.0113x; 1.0113x over previous
import jax
import jax.numpy as jnp
from jax import lax
from jax.experimental import pallas as pl
from jax.experimental.pallas import tpu as pltpu
from jax.experimental.pallas import tpu_sc as plsc

EMB_DIM = 128
HID_DIM = 256
BATCH = 16384

NUM_CORES = 2
NUM_SUBCORES = 16
NUM_WORKERS = NUM_CORES * NUM_SUBCORES

NCHUNK = 2
CHUNK = BATCH // NCHUNK
B_PER_W = CHUNK // NUM_WORKERS

BB = 2048
BLOCKS_PER_CHUNK = CHUNK // BB


def _gather_body(idx_hbm, table_hbm, out_hbm, idx_v, rows_v, sem):
    wid = lax.axis_index("s") * NUM_CORES + lax.axis_index("c")
    base = wid * B_PER_W
    pltpu.sync_copy(idx_hbm.at[pl.ds(base, B_PER_W)], idx_v)
    pltpu.async_copy(table_hbm.at[idx_v], rows_v, sem).wait()
    pltpu.sync_copy(rows_v, out_hbm.at[pl.ds(base, B_PER_W)])


def _fc_ln(x, w, b, g, be):
    h = jnp.dot(x, w, preferred_element_type=jnp.float32)
    h = jnp.maximum(h + b, 0.0)
    mean = jnp.mean(h, axis=-1, keepdims=True)
    var = jnp.mean(jnp.square(h - mean), axis=-1, keepdims=True)
    h_hat = (h - mean) * lax.rsqrt(var + 1e-5)
    return h_hat * g + be


def _fc_first_body(x_ref, w_ref, b_ref, g_ref, be_ref, o_ref):
    o_ref[...] = _fc_ln(x_ref[...], w_ref[...], b_ref[...], g_ref[...],
                        be_ref[...])


def _fc_second_body(x_ref, w_ref, b_ref, g_ref, be_ref, buf_ref, o_ref):
    del buf_ref
    o_ref[...] = _fc_ln(x_ref[...], w_ref[...], b_ref[...], g_ref[...],
                        be_ref[...])


def _make_gather():
    return pl.kernel(
        _gather_body,
        mesh=plsc.VectorSubcoreMesh(core_axis_name="c", subcore_axis_name="s"),
        out_type=jax.ShapeDtypeStruct((CHUNK, EMB_DIM), jnp.float32),
        scratch_types=[
            pltpu.VMEM((B_PER_W,), jnp.int32),
            pltpu.VMEM((B_PER_W, EMB_DIM), jnp.float32),
            pltpu.SemaphoreType.DMA,
        ],
    )


def kernel(item_input, table, W, b, gamma, beta):
    idx = item_input.astype(jnp.int32)
    b2 = b.reshape(1, HID_DIM)
    g2 = gamma.reshape(1, HID_DIM)
    be2 = beta.reshape(1, HID_DIM)

    gather = _make_gather()
    x0 = gather(idx[:CHUNK], table)
    x1 = gather(idx[CHUNK:], table)

    w_spec = pl.BlockSpec((EMB_DIM, HID_DIM), lambda i: (0, 0))
    v_spec = pl.BlockSpec((1, HID_DIM), lambda i: (0, 0))
    x_spec = pl.BlockSpec((BB, EMB_DIM), lambda i: (i, 0))

    out_a = pl.pallas_call(
        _fc_first_body,
        grid=(BLOCKS_PER_CHUNK,),
        in_specs=[x_spec, w_spec, v_spec, v_spec, v_spec],
        out_specs=pl.BlockSpec((BB, HID_DIM), lambda i: (i, 0)),
        out_shape=jax.ShapeDtypeStruct((BATCH, HID_DIM), jnp.float32),
    )(x0, W, b2, g2, be2)

    out = pl.pallas_call(
        _fc_second_body,
        grid=(BLOCKS_PER_CHUNK,),
        in_specs=[x_spec, w_spec, v_spec, v_spec, v_spec,
                  pl.BlockSpec(memory_space=pl.ANY)],
        out_specs=pl.BlockSpec(
            (BB, HID_DIM), lambda i: (i + BLOCKS_PER_CHUNK, 0)),
        out_shape=jax.ShapeDtypeStruct((BATCH, HID_DIM), jnp.float32),
        input_output_aliases={5: 0},
    )(x1, W, b2, g2, be2, out_a)
    return out


# speedup vs baseline: 1.0168x; 1.0055x over previous
import jax
import jax.numpy as jnp
from jax import lax
from jax.experimental import pallas as pl
from jax.experimental.pallas import tpu as pltpu
from jax.experimental.pallas import tpu_sc as plsc

EMB_DIM = 128
HID_DIM = 256
BATCH = 16384

NUM_CORES = 2
NUM_SUBCORES = 16
NUM_WORKERS = NUM_CORES * NUM_SUBCORES

NCHUNK = 2
CHUNK = BATCH // NCHUNK
B_PER_W = CHUNK // NUM_WORKERS

BB = 2048
BLOCKS_PER_CHUNK = CHUNK // BB


def _make_gather_body(chunk_id):
    def body(idx_hbm, table_hbm, out_hbm, idx_v, rows_v, sem):
        wid = lax.axis_index("s") * NUM_CORES + lax.axis_index("c")
        src = chunk_id * CHUNK + wid * B_PER_W
        dst = wid * B_PER_W
        pltpu.sync_copy(idx_hbm.at[pl.ds(src, B_PER_W)], idx_v)
        pltpu.async_copy(table_hbm.at[idx_v], rows_v, sem).wait()
        pltpu.sync_copy(rows_v, out_hbm.at[pl.ds(dst, B_PER_W)])
    return body


def _fc_ln(x, w, b, g, be):
    h = jnp.dot(x, w, preferred_element_type=jnp.float32)
    h = jnp.maximum(h + b, 0.0)
    mean = jnp.mean(h, axis=-1, keepdims=True)
    mean_sq = jnp.mean(jnp.square(h), axis=-1, keepdims=True)
    var = mean_sq - jnp.square(mean)
    r = lax.rsqrt(var + 1e-5)
    return (h - mean) * (r * g) + be


def _fc_first_body(x_ref, w_ref, b_ref, g_ref, be_ref, o_ref):
    o_ref[...] = _fc_ln(x_ref[...], w_ref[...], b_ref[...], g_ref[...],
                        be_ref[...])


def _fc_second_body(x_ref, w_ref, b_ref, g_ref, be_ref, buf_ref, o_ref):
    del buf_ref
    o_ref[...] = _fc_ln(x_ref[...], w_ref[...], b_ref[...], g_ref[...],
                        be_ref[...])


def _make_gather(chunk_id):
    return pl.kernel(
        _make_gather_body(chunk_id),
        mesh=plsc.VectorSubcoreMesh(core_axis_name="c", subcore_axis_name="s"),
        out_type=jax.ShapeDtypeStruct((CHUNK, EMB_DIM), jnp.float32),
        scratch_types=[
            pltpu.VMEM((B_PER_W,), jnp.int32),
            pltpu.VMEM((B_PER_W, EMB_DIM), jnp.float32),
            pltpu.SemaphoreType.DMA,
        ],
    )


def kernel(item_input, table, W, b, gamma, beta):
    idx = item_input.astype(jnp.int32)
    b2 = b.reshape(1, HID_DIM)
    g2 = gamma.reshape(1, HID_DIM)
    be2 = beta.reshape(1, HID_DIM)

    x0 = _make_gather(0)(idx, table)
    x1 = _make_gather(1)(idx, table)

    w_spec = pl.BlockSpec((EMB_DIM, HID_DIM), lambda i: (0, 0))
    v_spec = pl.BlockSpec((1, HID_DIM), lambda i: (0, 0))
    x_spec = pl.BlockSpec((BB, EMB_DIM), lambda i: (i, 0))

    out_a = pl.pallas_call(
        _fc_first_body,
        grid=(BLOCKS_PER_CHUNK,),
        in_specs=[x_spec, w_spec, v_spec, v_spec, v_spec],
        out_specs=pl.BlockSpec((BB, HID_DIM), lambda i: (i, 0)),
        out_shape=jax.ShapeDtypeStruct((BATCH, HID_DIM), jnp.float32),
    )(x0, W, b2, g2, be2)

    out = pl.pallas_call(
        _fc_second_body,
        grid=(BLOCKS_PER_CHUNK,),
        in_specs=[x_spec, w_spec, v_spec, v_spec, v_spec,
                  pl.BlockSpec(memory_space=pl.ANY)],
        out_specs=pl.BlockSpec(
            (BB, HID_DIM), lambda i: (i + BLOCKS_PER_CHUNK, 0)),
        out_shape=jax.ShapeDtypeStruct((BATCH, HID_DIM), jnp.float32),
        input_output_aliases={5: 0},
    )(x1, W, b2, g2, be2, out_a)
    return out
